# baseline (speedup 1.0000x reference)
import jax
import jax.numpy as jnp
from jax import lax
from jax.experimental import pallas as pl
from jax.experimental.pallas import tpu as pltpu
from jax.experimental.pallas import tpu_sc as plsc

_B = 8
_BSC = 4
_T = 4
_P = 1025
_H = 1280
_PH = _P * _H
_PR = 16
_HPR = _H // 16


def _sc_body(hid, ids, gate16, emb, table, out,
             embv, hva, hvb, tva, tvb, gv, idv,
             hsa, tsa, osa, hsb, tsb, osb):
    c = lax.axis_index("c")
    s = lax.axis_index("s")
    wid = s * 2 + c

    pltpu.sync_copy(ids, idv)
    pltpu.sync_copy(gate16, gv)
    g = 1.0 - 2.0 / (jnp.exp(2.0 * gv[...]) + 1.0)
    one_m_g = 1.0 - g
    nslab = _BSC * _T

    def start_in(i, hv, tv, hsem, tsem, p0, rows, cw):
        b = i // _T
        t = i % _T
        pltpu.make_async_copy(
            hid.at[b, t, pl.ds(p0, rows), :], hv.at[pl.ds(0, rows), :], hsem
        ).start()
        pltpu.make_async_copy(
            table.at[idv.at[pl.ds(b * 8, 1)], pl.ds(t * _PH + p0 * _H, cw)],
            tv.at[:, pl.ds(0, cw)], tsem).start()

    def wait_in(i, hv, tv, hsem, tsem, p0, rows, cw):
        b = i // _T
        t = i % _T
        pltpu.make_async_copy(
            hid.at[b, t, pl.ds(p0, rows), :], hv.at[pl.ds(0, rows), :], hsem
        ).wait()
        pltpu.make_async_copy(
            table.at[idv.at[pl.ds(b * 8, 1)], pl.ds(t * _PH + p0 * _H, cw)],
            tv.at[:, pl.ds(0, cw)], tsem).wait()

    def compute(hv, tv, rows):
        def row_body(row, _):
            base = row * _H

            @plsc.parallel_loop(0, _H, step=16, unroll=8)
            def _(col):
                hv[row, pl.ds(col, 16)] = (
                    hv[row, pl.ds(col, 16)]
                    + embv[row, pl.ds(col, 16)]
                    + g * tv[0, pl.ds(base + col, 16)])

            return 0
        lax.fori_loop(0, rows, row_body, 0)

    def out_copy(i, hv, osem, p0, rows):
        b = i // _T
        t = i % _T
        return pltpu.make_async_copy(
            hv.at[pl.ds(0, rows), :], out.at[b, t, pl.ds(p0, rows), :], osem)

    def do_slice(si, rows):
        cw = rows * _H
        p0 = si * _PR
        pltpu.sync_copy(emb.at[pl.ds(p0, rows), :], embv.at[pl.ds(0, rows), :])

        def escale(row, _):
            @plsc.parallel_loop(0, _H, step=16, unroll=8)
            def _(col):
                embv[row, pl.ds(col, 16)] = one_m_g * embv[row, pl.ds(col, 16)]
            return 0
        lax.fori_loop(0, rows, escale, 0)

        start_in(0, hva, tva, hsa, tsa, p0, rows, cw)

        def pair(j, _):
            i0 = 2 * j
            i1 = 2 * j + 1

            @pl.when(j > 0)
            def _():
                out_copy(i1 - 2, hvb, osb, p0, rows).wait()
            start_in(i1, hvb, tvb, hsb, tsb, p0, rows, cw)

            wait_in(i0, hva, tva, hsa, tsa, p0, rows, cw)
            compute(hva, tva, rows)
            out_copy(i0, hva, osa, p0, rows).start()

            @pl.when(j < nslab // 2 - 1)
            def _():
                out_copy(i0, hva, osa, p0, rows).wait()
                start_in(i0 + 2, hva, tva, hsa, tsa, p0, rows, cw)

            wait_in(i1, hvb, tvb, hsb, tsb, p0, rows, cw)
            compute(hvb, tvb, rows)
            out_copy(i1, hvb, osb, p0, rows).start()
            return 0

        lax.fori_loop(0, nslab // 2, pair, 0)
        out_copy(nslab - 2, hva, osa, p0, rows).wait()
        out_copy(nslab - 1, hvb, osb, p0, rows).wait()

    do_slice(wid, _PR)
    do_slice(wid + 32, _PR)

    @pl.when(wid == 0)
    def _():
        do_slice(64, 1)


def _tc_body(ids_ref, h_ref, t_ref, e_ref, gate_ref, o_ref):
    g = jnp.tanh(gate_ref[0])
    tt = t_ref[...].reshape(_P, _H)
    o_ref[...] = (h_ref[...]
                  + (1.0 - g) * e_ref[...]
                  + g * tt[None, None])


def kernel(hidden_state, aspect_ratio_ids, gate, embedding, tile_embedding_weight):
    ids32 = aspect_ratio_ids.astype(jnp.int32)
    ids = jnp.zeros((_B * 8,), jnp.int32).at[::8].set(ids32)
    gate16 = jnp.broadcast_to(gate, (16,))

    mesh = plsc.VectorSubcoreMesh(core_axis_name="c", subcore_axis_name="s")
    sc = pl.kernel(
        _sc_body,
        out_type=jax.ShapeDtypeStruct((_BSC, _T, _P, _H), jnp.float32),
        mesh=mesh,
        scratch_types=[
            pltpu.VMEM((_PR, _H), jnp.float32),
            pltpu.VMEM((_PR, _H), jnp.float32),
            pltpu.VMEM((_PR, _H), jnp.float32),
            pltpu.VMEM((1, _PR * _H), jnp.float32),
            pltpu.VMEM((1, _PR * _H), jnp.float32),
            pltpu.VMEM((16,), jnp.float32),
            pltpu.VMEM((_B * 8,), jnp.int32),
            pltpu.SemaphoreType.DMA,
            pltpu.SemaphoreType.DMA,
            pltpu.SemaphoreType.DMA,
            pltpu.SemaphoreType.DMA,
            pltpu.SemaphoreType.DMA,
            pltpu.SemaphoreType.DMA,
        ],
        compiler_params=pltpu.CompilerParams(use_tc_tiling_on_sc=True),
    )
    sc_out = sc(hidden_state, ids, gate16, embedding, tile_embedding_weight)

    tv = tile_embedding_weight.reshape(9, 1, _T * _PH)
    grid_spec = pltpu.PrefetchScalarGridSpec(
        num_scalar_prefetch=1,
        grid=(_B - _BSC, _T),
        in_specs=[
            pl.BlockSpec((1, 1, _P, _H), lambda b, t, ids: (b + _BSC, t, 0, 0)),
            pl.BlockSpec((1, 1, _PH), lambda b, t, ids: (ids[b + _BSC], 0, t)),
            pl.BlockSpec((_P, _H), lambda b, t, ids: (0, 0)),
            pl.BlockSpec(memory_space=pltpu.SMEM),
        ],
        out_specs=pl.BlockSpec((1, 1, _P, _H), lambda b, t, ids: (b + _BSC, t, 0, 0)),
    )
    tc_out = pl.pallas_call(
        _tc_body,
        grid_spec=grid_spec,
        out_shape=jax.ShapeDtypeStruct((_B, _T, _P, _H), jnp.float32),
        compiler_params=pltpu.CompilerParams(
            dimension_semantics=("arbitrary", "arbitrary"),
        ),
    )(ids32, hidden_state, tv, embedding, gate)

    return lax.dynamic_update_slice(tc_out, sc_out, (0, 0, 0, 0))


# speedup vs baseline: 1.8535x; 1.8535x over previous
import jax
import jax.numpy as jnp
from jax import lax
from jax.experimental import pallas as pl
from jax.experimental.pallas import tpu as pltpu
from jax.experimental.pallas import tpu_sc as plsc

_B = 8
_BSC = 4
_T = 4
_P = 1025
_H = 1280
_PH = _P * _H
_PR = 16
_HPR = _H // 16


def _sc_body(hid, ids, gate16, emb, table, out,
             embv, hva, hvb, tva, tvb, gv, idv,
             hsa, tsa, osa, hsb, tsb, osb):
    c = lax.axis_index("c")
    s = lax.axis_index("s")
    wid = s * 2 + c

    pltpu.sync_copy(ids, idv)
    pltpu.sync_copy(gate16, gv)
    g = 1.0 - 2.0 / (jnp.exp(2.0 * gv[...]) + 1.0)
    one_m_g = 1.0 - g
    nslab = _BSC * _T

    def start_in(i, hv, tv, hsem, tsem, p0, rows, cw):
        b = i // _T
        t = i % _T
        pltpu.make_async_copy(
            hid.at[b, t, pl.ds(p0, rows), :], hv.at[pl.ds(0, rows), :], hsem
        ).start()
        pltpu.make_async_copy(
            table.at[idv.at[pl.ds(b * 8, 1)], pl.ds(t * _PH + p0 * _H, cw)],
            tv.at[:, pl.ds(0, cw)], tsem).start()

    def wait_in(i, hv, tv, hsem, tsem, p0, rows, cw):
        b = i // _T
        t = i % _T
        pltpu.make_async_copy(
            hid.at[b, t, pl.ds(p0, rows), :], hv.at[pl.ds(0, rows), :], hsem
        ).wait()
        pltpu.make_async_copy(
            table.at[idv.at[pl.ds(b * 8, 1)], pl.ds(t * _PH + p0 * _H, cw)],
            tv.at[:, pl.ds(0, cw)], tsem).wait()

    def compute(hv, tv, rows):
        def row_body(row, _):
            base = row * _H

            @plsc.parallel_loop(0, _H, step=16, unroll=8)
            def _(col):
                hv[row, pl.ds(col, 16)] = (
                    hv[row, pl.ds(col, 16)]
                    + embv[row, pl.ds(col, 16)]
                    + g * tv[0, pl.ds(base + col, 16)])

            return 0
        lax.fori_loop(0, rows, row_body, 0)

    def out_copy(i, hv, osem, p0, rows):
        b = i // _T
        t = i % _T
        return pltpu.make_async_copy(
            hv.at[pl.ds(0, rows), :], out.at[b, t, pl.ds(p0, rows), :], osem)

    def do_slice(si, rows):
        cw = rows * _H
        p0 = si * _PR
        pltpu.sync_copy(emb.at[pl.ds(p0, rows), :], embv.at[pl.ds(0, rows), :])

        def escale(row, _):
            @plsc.parallel_loop(0, _H, step=16, unroll=8)
            def _(col):
                embv[row, pl.ds(col, 16)] = one_m_g * embv[row, pl.ds(col, 16)]
            return 0
        lax.fori_loop(0, rows, escale, 0)

        start_in(0, hva, tva, hsa, tsa, p0, rows, cw)

        def pair(j, _):
            i0 = 2 * j
            i1 = 2 * j + 1

            @pl.when(j > 0)
            def _():
                out_copy(i1 - 2, hvb, osb, p0, rows).wait()
            start_in(i1, hvb, tvb, hsb, tsb, p0, rows, cw)

            wait_in(i0, hva, tva, hsa, tsa, p0, rows, cw)
            compute(hva, tva, rows)
            out_copy(i0, hva, osa, p0, rows).start()

            @pl.when(j < nslab // 2 - 1)
            def _():
                out_copy(i0, hva, osa, p0, rows).wait()
                start_in(i0 + 2, hva, tva, hsa, tsa, p0, rows, cw)

            wait_in(i1, hvb, tvb, hsb, tsb, p0, rows, cw)
            compute(hvb, tvb, rows)
            out_copy(i1, hvb, osb, p0, rows).start()
            return 0

        lax.fori_loop(0, nslab // 2, pair, 0)
        out_copy(nslab - 2, hva, osa, p0, rows).wait()
        out_copy(nslab - 1, hvb, osb, p0, rows).wait()

    do_slice(wid, _PR)
    do_slice(wid + 32, _PR)

    @pl.when(wid == 0)
    def _():
        do_slice(64, 1)


def _tc_body(ids_ref, h_ref, t_ref, e_ref, gate_ref, sc_ref, o_ref):
    g = jnp.tanh(gate_ref[0])
    tt = t_ref[...].reshape(_P, _H)
    o_ref[...] = (h_ref[...]
                  + (1.0 - g) * e_ref[...]
                  + g * tt[None, None])


def kernel(hidden_state, aspect_ratio_ids, gate, embedding, tile_embedding_weight):
    ids32 = aspect_ratio_ids.astype(jnp.int32)
    ids = jnp.zeros((_B * 8,), jnp.int32).at[::8].set(ids32)
    gate16 = jnp.broadcast_to(gate, (16,))

    mesh = plsc.VectorSubcoreMesh(core_axis_name="c", subcore_axis_name="s")
    sc = pl.kernel(
        _sc_body,
        out_type=jax.ShapeDtypeStruct((_B, _T, _P, _H), jnp.float32),
        mesh=mesh,
        scratch_types=[
            pltpu.VMEM((_PR, _H), jnp.float32),
            pltpu.VMEM((_PR, _H), jnp.float32),
            pltpu.VMEM((_PR, _H), jnp.float32),
            pltpu.VMEM((1, _PR * _H), jnp.float32),
            pltpu.VMEM((1, _PR * _H), jnp.float32),
            pltpu.VMEM((16,), jnp.float32),
            pltpu.VMEM((_B * 8,), jnp.int32),
            pltpu.SemaphoreType.DMA,
            pltpu.SemaphoreType.DMA,
            pltpu.SemaphoreType.DMA,
            pltpu.SemaphoreType.DMA,
            pltpu.SemaphoreType.DMA,
            pltpu.SemaphoreType.DMA,
        ],
        compiler_params=pltpu.CompilerParams(use_tc_tiling_on_sc=True),
    )
    sc_out = sc(hidden_state, ids, gate16, embedding, tile_embedding_weight)

    tv = tile_embedding_weight.reshape(9, 1, _T * _PH)
    grid_spec = pltpu.PrefetchScalarGridSpec(
        num_scalar_prefetch=1,
        grid=(_B - _BSC, _T),
        in_specs=[
            pl.BlockSpec((1, 1, _P, _H), lambda b, t, ids: (b + _BSC, t, 0, 0)),
            pl.BlockSpec((1, 1, _PH), lambda b, t, ids: (ids[b + _BSC], 0, t)),
            pl.BlockSpec((_P, _H), lambda b, t, ids: (0, 0)),
            pl.BlockSpec(memory_space=pltpu.SMEM),
            pl.BlockSpec(memory_space=pltpu.HBM),
        ],
        out_specs=pl.BlockSpec((1, 1, _P, _H), lambda b, t, ids: (b + _BSC, t, 0, 0)),
    )
    tc_out = pl.pallas_call(
        _tc_body,
        grid_spec=grid_spec,
        out_shape=jax.ShapeDtypeStruct((_B, _T, _P, _H), jnp.float32),
        input_output_aliases={5: 0},
        compiler_params=pltpu.CompilerParams(
            dimension_semantics=("arbitrary", "arbitrary"),
        ),
    )(ids32, hidden_state, tv, embedding, gate, sc_out)

    return tc_out


# speedup vs baseline: 2.2933x; 1.2373x over previous
import jax
import jax.numpy as jnp
from jax import lax
from jax.experimental import pallas as pl
from jax.experimental.pallas import tpu as pltpu
from jax.experimental.pallas import tpu_sc as plsc

_B = 8
_T = 4
_P = 1025
_H = 1280
_PH = _P * _H
_PR = 8
_CW = _PR * _H
_NSLAB = _B * _T
_DEPTH = 4


def _sc_body(hid, ids, gate16, emb, table, out,
             embv, hv0, hv1, hv2, hv3, tv0, tv1, tv2, tv3, gv, idv,
             hs0, hs1, hs2, hs3, ts0, ts1, ts2, ts3, os0, os1, os2, os3):
    c = lax.axis_index("c")
    s = lax.axis_index("s")
    wid = s * 2 + c

    hv = [hv0, hv1, hv2, hv3]
    tv = [tv0, tv1, tv2, tv3]
    hs = [hs0, hs1, hs2, hs3]
    ts = [ts0, ts1, ts2, ts3]
    os_ = [os0, os1, os2, os3]

    pltpu.sync_copy(ids, idv)
    pltpu.sync_copy(gate16, gv)
    g = 1.0 - 2.0 / (jnp.exp(2.0 * gv[...]) + 1.0)
    one_m_g = 1.0 - g

    def h_copy(i, m, p0, rows):
        b = i // _T
        t = i % _T
        return pltpu.make_async_copy(
            hid.at[b, t, pl.ds(p0, rows), :], hv[m].at[pl.ds(0, rows), :], hs[m])

    def t_copy(i, m, p0, rows):
        b = i // _T
        t = i % _T
        return pltpu.make_async_copy(
            table.at[idv.at[pl.ds(b * 8, 1)],
                     pl.ds(t * _PH + p0 * _H, rows * _H)],
            tv[m].at[:, pl.ds(0, rows * _H)], ts[m])

    def o_copy(i, m, p0, rows):
        b = i // _T
        t = i % _T
        return pltpu.make_async_copy(
            hv[m].at[pl.ds(0, rows), :], out.at[b, t, pl.ds(p0, rows), :], os_[m])

    def compute(m, rows):
        def row_body(row, _):
            base = row * _H

            @plsc.parallel_loop(0, _H, step=16, unroll=8)
            def _(col):
                hv[m][row, pl.ds(col, 16)] = (
                    hv[m][row, pl.ds(col, 16)]
                    + embv[row, pl.ds(col, 16)]
                    + g * tv[m][0, pl.ds(base + col, 16)])

            return 0
        lax.fori_loop(0, rows, row_body, 0)

    def load_emb(p0, rows):
        pltpu.sync_copy(emb.at[pl.ds(p0, rows), :], embv.at[pl.ds(0, rows), :])

        def escale(row, _):
            @plsc.parallel_loop(0, _H, step=16, unroll=8)
            def _(col):
                embv[row, pl.ds(col, 16)] = one_m_g * embv[row, pl.ds(col, 16)]
            return 0
        lax.fori_loop(0, rows, escale, 0)

    def do_slice(si):
        p0 = si * _PR
        load_emb(p0, _PR)

        h_copy(0, 0, p0, _PR).start()
        t_copy(0, 0, p0, _PR).start()
        h_copy(1, 1, p0, _PR).start()
        t_copy(1, 1, p0, _PR).start()

        def quad(q, _):
            for m in range(_DEPTH):
                i = 4 * q + m
                h_copy(i, m, p0, _PR).wait()
                t_copy(i, m, p0, _PR).wait()
                compute(m, _PR)
                o_copy(i, m, p0, _PR).start()
                mp = (m + 2) % _DEPTH
                if m < 2:
                    @pl.when(q > 0)
                    def _():
                        o_copy(i - 2, mp, p0, _PR).wait()
                    h_copy(i + 2, mp, p0, _PR).start()
                    t_copy(i + 2, mp, p0, _PR).start()
                else:
                    @pl.when(q < _NSLAB // 4 - 1)
                    def _():
                        o_copy(i - 2, mp, p0, _PR).wait()
                        h_copy(i + 2, mp, p0, _PR).start()
                        t_copy(i + 2, mp, p0, _PR).start()
            return 0

        lax.fori_loop(0, _NSLAB // 4, quad, 0)
        for m in range(_DEPTH):
            o_copy(_NSLAB - 4 + m, m, p0, _PR).wait()

    for k in range(4):
        do_slice(wid + 32 * k)

    p0 = _P - 1
    load_emb(p0, 1)
    h_copy(wid, 0, p0, 1).start()
    t_copy(wid, 0, p0, 1).start()
    h_copy(wid, 0, p0, 1).wait()
    t_copy(wid, 0, p0, 1).wait()
    compute(0, 1)
    o_copy(wid, 0, p0, 1).start()
    o_copy(wid, 0, p0, 1).wait()


def kernel(hidden_state, aspect_ratio_ids, gate, embedding, tile_embedding_weight):
    ids = jnp.zeros((_B * 8,), jnp.int32).at[::8].set(
        aspect_ratio_ids.astype(jnp.int32))
    gate16 = jnp.broadcast_to(gate, (16,))

    mesh = plsc.VectorSubcoreMesh(core_axis_name="c", subcore_axis_name="s")
    sc = pl.kernel(
        _sc_body,
        out_type=jax.ShapeDtypeStruct((_B, _T, _P, _H), jnp.float32),
        mesh=mesh,
        scratch_types=(
            [pltpu.VMEM((_PR, _H), jnp.float32)]
            + [pltpu.VMEM((_PR, _H), jnp.float32)] * 4
            + [pltpu.VMEM((1, _CW), jnp.float32)] * 4
            + [pltpu.VMEM((16,), jnp.float32)]
            + [pltpu.VMEM((_B * 8,), jnp.int32)]
            + [pltpu.SemaphoreType.DMA] * 12
        ),
        compiler_params=pltpu.CompilerParams(use_tc_tiling_on_sc=True),
    )
    return sc(hidden_state, ids, gate16, embedding, tile_embedding_weight)
